# Initial kernel scaffold; baseline (speedup 1.0000x reference)
import jax
import jax.numpy as jnp
from jax.experimental import pallas as pl


def kernel(x, edge_index, edge_type, bases1, comb1, root1, bias1, g1, b1, bases2, comb2, root2, bias2, g2, b2):
    raise NotImplementedError("write your pallas kernel here")


# speedup vs baseline: 3.9299x; 3.9299x over previous
import functools

import jax
import jax.numpy as jnp
from jax import lax
from jax.experimental import pallas as pl
from jax.experimental.pallas import tpu as pltpu

N = 10000
E = 320000
D = 128
R = 9
EPS = 1e-5

BN = 1000


def _tc_layer1(x_ref, comb_ref, bases_ref, root_ref, bias_ref,
               xr_ref, hroot_ref):
    W = lax.dot_general(comb_ref[...], bases_ref[...],
                        (((1,), (0,)), ((), ())),
                        preferred_element_type=jnp.float32)
    xb = x_ref[...]
    for r in range(R):
        xr_ref[r] = jnp.dot(xb, W[r], preferred_element_type=jnp.float32)
    hroot_ref[...] = (jnp.dot(xb, root_ref[...],
                              preferred_element_type=jnp.float32)
                      + bias_ref[...][None, :])


def _tc_mid(agg_ref, hroot_ref, g_ref, b_ref, comb_ref, bases_ref,
            root_ref, bias_ref, xr_ref, hroot2_ref):
    a = agg_ref[...]
    a = jnp.where(jnp.isneginf(a), 0.0, a)
    h = a + hroot_ref[...]
    mu = jnp.mean(h, axis=-1, keepdims=True)
    var = jnp.mean((h - mu) ** 2, axis=-1, keepdims=True)
    h = (h - mu) * lax.rsqrt(var + EPS) * g_ref[...][None, :] + b_ref[...][None, :]
    h = jnp.maximum(h, 0.0)
    W = lax.dot_general(comb_ref[...], bases_ref[...],
                        (((1,), (0,)), ((), ())),
                        preferred_element_type=jnp.float32)
    for r in range(R):
        xr_ref[r] = jnp.dot(h, W[r], preferred_element_type=jnp.float32)
    hroot2_ref[...] = (jnp.dot(h, root_ref[...],
                               preferred_element_type=jnp.float32)
                       + bias_ref[...][None, :])


def _tc_final(agg0_ref, agg1_ref, hroot_ref, g_ref, b_ref, out_ref):
    h = agg0_ref[...] + agg1_ref[...] + hroot_ref[...]
    mu = jnp.mean(h, axis=-1, keepdims=True)
    var = jnp.mean((h - mu) ** 2, axis=-1, keepdims=True)
    h = (h - mu) * lax.rsqrt(var + EPS) * g_ref[...][None, :] + b_ref[...][None, :]
    hmax = jnp.max(h, axis=-1, keepdims=True)
    z = h - hmax
    lse = jnp.log(jnp.sum(jnp.exp(z), axis=-1, keepdims=True))
    out_ref[...] = z - lse


def _tc_gidx(src_ref, et_ref, gidx_ref):
    gidx_ref[...] = et_ref[...] * N + src_ref[...]


_row_spec = pl.BlockSpec((BN, D), lambda i: (i, 0))
_full = lambda shape: pl.BlockSpec(shape, lambda i: (0,) * len(shape))
_xr_spec = pl.BlockSpec((R, BN, D), lambda i: (0, i, 0))


def _layer1_call(x, comb, bases, root, bias):
    return pl.pallas_call(
        _tc_layer1,
        grid=(N // BN,),
        in_specs=[_row_spec, _full((R, R)), _full((R, D, D)),
                  _full((D, D)), _full((D,))],
        out_specs=[_xr_spec, _row_spec],
        out_shape=[jax.ShapeDtypeStruct((R, N, D), jnp.float32),
                   jax.ShapeDtypeStruct((N, D), jnp.float32)],
    )(x, comb, bases, root, bias)


def _mid_call(agg, hroot, g, b, comb, bases, root, bias):
    return pl.pallas_call(
        _tc_mid,
        grid=(N // BN,),
        in_specs=[_row_spec, _row_spec, _full((D,)), _full((D,)),
                  _full((R, R)), _full((R, D, D)), _full((D, D)), _full((D,))],
        out_specs=[_xr_spec, _row_spec],
        out_shape=[jax.ShapeDtypeStruct((R, N, D), jnp.float32),
                   jax.ShapeDtypeStruct((N, D), jnp.float32)],
    )(agg, hroot, g, b, comb, bases, root, bias)


def _final_call(agg0, agg1, hroot, g, b):
    return pl.pallas_call(
        _tc_final,
        grid=(N // BN,),
        in_specs=[_row_spec, _row_spec, _row_spec, _full((D,)), _full((D,))],
        out_specs=_row_spec,
        out_shape=jax.ShapeDtypeStruct((N, D), jnp.float32),
    )(agg0, agg1, hroot, g, b)


def _gidx_call(src, et):
    s2 = src.reshape(2500, 128)
    e2 = et.reshape(2500, 128)
    return pl.pallas_call(
        _tc_gidx,
        in_specs=[pl.BlockSpec((2500, 128), lambda: (0, 0))] * 2,
        out_specs=pl.BlockSpec((2500, 128), lambda: (0, 0)),
        out_shape=jax.ShapeDtypeStruct((2500, 128), jnp.int32),
    )(s2, e2).reshape(E)


def kernel(x, edge_index, edge_type, bases1, comb1, root1, bias1, g1, b1,
           bases2, comb2, root2, bias2, g2, b2):
    src = edge_index[0]
    dst = edge_index[1]
    gidx = _gidx_call(src, edge_type)

    xr1, hroot1 = _layer1_call(x, comb1, bases1, root1, bias1)
    xr1f = xr1.reshape(R * N, D)

    m1 = xr1f[gidx]
    agg1 = jax.ops.segment_max(m1, dst, num_segments=N)

    xr2, hroot2 = _mid_call(agg1, hroot1, g1, b1, comb2, bases2, root2, bias2)
    xr2f = xr2.reshape(R * N, D)

    m2 = xr2f[gidx]
    agg2 = jax.ops.segment_sum(m2, dst, num_segments=N)

    return _final_call(agg2, jnp.zeros_like(agg2), hroot2, g2, b2)


# speedup vs baseline: 4.6565x; 1.1849x over previous
import functools

import jax
import jax.numpy as jnp
from jax import lax
from jax.experimental import pallas as pl
from jax.experimental.pallas import tpu as pltpu
from jax.experimental.pallas import tpu_sc as plsc

N = 10000
E = 320000
D = 128
R = 9
EPS = 1e-5

BN = 1000

NSC = 2
NSS = 16
NW = NSC * NSS
EPW = E // NW
CH = 128
NCH = EPW // CH
TAIL = EPW - NCH * CH
NPS = 624

_sc_mesh = plsc.VectorSubcoreMesh(core_axis_name="c", subcore_axis_name="s")


def _sc_sum_body(xr_hbm, gidx_hbm, dst_hbm, zeros_hbm, out_hbm,
                 gix_v, dst_v, tg_v, td_v, rows_v, acc_sh, sem):
    c = lax.axis_index("c")
    s = lax.axis_index("s")
    w = c * NSS + s
    ebase = w * EPW
    pltpu.sync_copy(zeros_hbm.at[pl.ds(0, NPS)], acc_sh.at[pl.ds(s * NPS, NPS)])

    @pl.when(s == 0)
    def _():
        pltpu.sync_copy(zeros_hbm.at[pl.ds(0, N - NSS * NPS)],
                        acc_sh.at[pl.ds(NSS * NPS, N - NSS * NPS)])

    plsc.subcore_barrier()

    def body(k, carry):
        off = ebase + k * CH
        pltpu.sync_copy(gidx_hbm.at[pl.ds(off, CH)], gix_v)
        pltpu.sync_copy(dst_hbm.at[pl.ds(off, CH)], dst_v)
        pltpu.async_copy(xr_hbm.at[gix_v], rows_v, sem).wait()
        pltpu.sync_copy(rows_v, acc_sh.at[dst_v], add=True)
        return carry

    lax.fori_loop(0, NCH, body, 0)
    toff = ebase + NCH * CH
    pltpu.sync_copy(gidx_hbm.at[pl.ds(toff, TAIL)], tg_v)
    pltpu.sync_copy(dst_hbm.at[pl.ds(toff, TAIL)], td_v)
    pltpu.async_copy(xr_hbm.at[tg_v], rows_v.at[pl.ds(0, TAIL)], sem).wait()
    pltpu.sync_copy(rows_v.at[pl.ds(0, TAIL)], acc_sh.at[td_v], add=True)

    plsc.subcore_barrier()
    pltpu.sync_copy(acc_sh.at[pl.ds(s * NPS, NPS)],
                    out_hbm.at[c].at[pl.ds(s * NPS, NPS)])

    @pl.when(s == 0)
    def _():
        pltpu.sync_copy(acc_sh.at[pl.ds(NSS * NPS, N - NSS * NPS)],
                        out_hbm.at[c].at[pl.ds(NSS * NPS, N - NSS * NPS)])


ROWS_PT = 313
NOUT = NW * ROWS_PT
C2 = 512
NC2 = E // C2
CAP = C2 + 16
ACCL = (ROWS_PT + 1) * D


def _sc_max_body(xr_hbm, gidx_hbm, dst_hbm, out_hbm,
                 dstb, gixb, mbg, mbd, rows3, accf, sem):
    c = lax.axis_index("c")
    s = lax.axis_index("s")
    w = c * NSS + s
    lo = w * ROWS_PT
    neg = jnp.full((16,), -jnp.inf, jnp.float32)

    def init_acc(i, carry):
        accf[pl.ds(i * 16, 16)] = neg
        return carry

    lax.fori_loop(0, ACCL // 16, init_acc, 0)

    dummy_g = jnp.zeros((16,), jnp.int32)
    dummy_d = jnp.full((16,), ROWS_PT, jnp.int32)

    def init_mb(i, carry):
        mbg[pl.ds(i * 16, 16)] = dummy_g
        mbd[pl.ds(i * 16, 16)] = dummy_d
        return carry

    lax.fori_loop(0, CAP // 16, init_mb, 0)

    iota = lax.iota(jnp.int32, 16)

    def chunk(k, carry):
        off = k * C2
        pltpu.sync_copy(dst_hbm.at[pl.ds(off, C2)], dstb)
        pltpu.sync_copy(gidx_hbm.at[pl.ds(off, C2)], gixb)
        base = jnp.zeros((16,), jnp.int32)
        for i in range(C2 // 16):
            dv = dstb[pl.ds(i * 16, 16)]
            gv = gixb[pl.ds(i * 16, 16)]
            ldv = dv - lo
            m = (ldv >= 0) & (ldv < ROWS_PT)
            pos = base + plsc.cumsum(m.astype(jnp.int32)) - 1
            plsc.store_scatter(mbg, [pos], gv, mask=m)
            plsc.store_scatter(mbd, [pos], ldv, mask=m)
            base = base + plsc.all_reduce_population_count(m)
        cnt = jnp.max(base)
        nb = (cnt + 15) // 16

        def drain(j, carry2):
            par = j % 2
            pltpu.async_copy(xr_hbm.at[mbg.at[pl.ds(j * 16, 16)]],
                             rows3.at[par], sem).wait()
            parsp = jnp.full((16,), par, jnp.int32)
            for e in range(16):
                ldsp = plsc.load_gather(mbd, [jnp.full((16,), j * 16 + e,
                                                       jnp.int32)])
                abase = ldsp * D + iota
                esp = jnp.full((16,), e, jnp.int32)
                for cg in range(D // 16):
                    ai = abase + (cg * 16)
                    a = plsc.load_gather(accf, [ai])
                    r = plsc.load_gather(rows3, [parsp, esp, (cg * 16) + iota])
                    plsc.store_scatter(accf, [ai], jnp.maximum(a, r))
            return carry2

        lax.fori_loop(0, nb, drain, 0)
        return carry

    lax.fori_loop(0, NC2, chunk, 0)
    pltpu.sync_copy(accf.at[pl.ds(0, ROWS_PT * D)], out_hbm.at[w])


@functools.partial(
    pl.kernel,
    out_type=jax.ShapeDtypeStruct((NW, ROWS_PT * D), jnp.float32),
    mesh=_sc_mesh,
    scratch_types=[
        pltpu.VMEM((C2,), jnp.int32),
        pltpu.VMEM((C2,), jnp.int32),
        pltpu.VMEM((CAP,), jnp.int32),
        pltpu.VMEM((CAP,), jnp.int32),
        pltpu.VMEM((2, 16, D), jnp.float32),
        pltpu.VMEM((ACCL,), jnp.float32),
        pltpu.SemaphoreType.DMA,
    ],
    compiler_params=pltpu.CompilerParams(needs_layout_passes=False),
)
def _sc_max(xr_hbm, gidx_hbm, dst_hbm, out_hbm,
            dstb, gixb, mbg, mbd, rows3, accf, sem):
    _sc_max_body(xr_hbm, gidx_hbm, dst_hbm, out_hbm,
                 dstb, gixb, mbg, mbd, rows3, accf, sem)


@functools.partial(
    pl.kernel,
    out_type=jax.ShapeDtypeStruct((NSC, N, D), jnp.float32),
    mesh=_sc_mesh,
    scratch_types=[
        pltpu.VMEM((CH,), jnp.int32),
        pltpu.VMEM((CH,), jnp.int32),
        pltpu.VMEM((TAIL,), jnp.int32),
        pltpu.VMEM((TAIL,), jnp.int32),
        pltpu.VMEM((CH, D), jnp.float32),
        pltpu.VMEM_SHARED((N, D), jnp.float32),
        pltpu.SemaphoreType.DMA,
    ],
)
def _sc_sum(xr_hbm, gidx_hbm, dst_hbm, zeros_hbm, out_hbm,
            gix_v, dst_v, tg_v, td_v, rows_v, acc_sh, sem):
    _sc_sum_body(xr_hbm, gidx_hbm, dst_hbm, zeros_hbm, out_hbm,
                 gix_v, dst_v, tg_v, td_v, rows_v, acc_sh, sem)


def _tc_layer1(x_ref, comb_ref, bases_ref, root_ref, bias_ref,
               xr_ref, hroot_ref):
    W = lax.dot_general(comb_ref[...], bases_ref[...],
                        (((1,), (0,)), ((), ())),
                        preferred_element_type=jnp.float32)
    xb = x_ref[...]
    for r in range(R):
        xr_ref[r] = jnp.dot(xb, W[r], preferred_element_type=jnp.float32)
    hroot_ref[...] = (jnp.dot(xb, root_ref[...],
                              preferred_element_type=jnp.float32)
                      + bias_ref[...][None, :])


def _tc_mid(agg_ref, hroot_ref, g_ref, b_ref, comb_ref, bases_ref,
            root_ref, bias_ref, xr_ref, hroot2_ref):
    a = agg_ref[...]
    a = jnp.where(jnp.isneginf(a), 0.0, a)
    h = a + hroot_ref[...]
    mu = jnp.mean(h, axis=-1, keepdims=True)
    var = jnp.mean((h - mu) ** 2, axis=-1, keepdims=True)
    h = (h - mu) * lax.rsqrt(var + EPS) * g_ref[...][None, :] + b_ref[...][None, :]
    h = jnp.maximum(h, 0.0)
    W = lax.dot_general(comb_ref[...], bases_ref[...],
                        (((1,), (0,)), ((), ())),
                        preferred_element_type=jnp.float32)
    for r in range(R):
        xr_ref[r] = jnp.dot(h, W[r], preferred_element_type=jnp.float32)
    hroot2_ref[...] = (jnp.dot(h, root_ref[...],
                               preferred_element_type=jnp.float32)
                       + bias_ref[...][None, :])


def _tc_final(agg0_ref, agg1_ref, hroot_ref, g_ref, b_ref, out_ref):
    h = agg0_ref[...] + agg1_ref[...] + hroot_ref[...]
    mu = jnp.mean(h, axis=-1, keepdims=True)
    var = jnp.mean((h - mu) ** 2, axis=-1, keepdims=True)
    h = (h - mu) * lax.rsqrt(var + EPS) * g_ref[...][None, :] + b_ref[...][None, :]
    hmax = jnp.max(h, axis=-1, keepdims=True)
    z = h - hmax
    lse = jnp.log(jnp.sum(jnp.exp(z), axis=-1, keepdims=True))
    out_ref[...] = z - lse


def _tc_gidx(src_ref, et_ref, gidx_ref):
    gidx_ref[...] = et_ref[...] * N + src_ref[...]


_row_spec = pl.BlockSpec((BN, D), lambda i: (i, 0))
_full = lambda shape: pl.BlockSpec(shape, lambda i: (0,) * len(shape))
_xr_spec = pl.BlockSpec((R, BN, D), lambda i: (0, i, 0))


def _layer1_call(x, comb, bases, root, bias):
    return pl.pallas_call(
        _tc_layer1,
        grid=(N // BN,),
        in_specs=[_row_spec, _full((R, R)), _full((R, D, D)),
                  _full((D, D)), _full((D,))],
        out_specs=[_xr_spec, _row_spec],
        out_shape=[jax.ShapeDtypeStruct((R, N, D), jnp.float32),
                   jax.ShapeDtypeStruct((N, D), jnp.float32)],
    )(x, comb, bases, root, bias)


def _mid_call(agg, hroot, g, b, comb, bases, root, bias):
    return pl.pallas_call(
        _tc_mid,
        grid=(N // BN,),
        in_specs=[_row_spec, _row_spec, _full((D,)), _full((D,)),
                  _full((R, R)), _full((R, D, D)), _full((D, D)), _full((D,))],
        out_specs=[_xr_spec, _row_spec],
        out_shape=[jax.ShapeDtypeStruct((R, N, D), jnp.float32),
                   jax.ShapeDtypeStruct((N, D), jnp.float32)],
    )(agg, hroot, g, b, comb, bases, root, bias)


def _final_call(agg0, agg1, hroot, g, b):
    return pl.pallas_call(
        _tc_final,
        grid=(N // BN,),
        in_specs=[_row_spec, _row_spec, _row_spec, _full((D,)), _full((D,))],
        out_specs=_row_spec,
        out_shape=jax.ShapeDtypeStruct((N, D), jnp.float32),
    )(agg0, agg1, hroot, g, b)


def _gidx_call(src, et):
    s2 = src.reshape(2500, 128)
    e2 = et.reshape(2500, 128)
    return pl.pallas_call(
        _tc_gidx,
        in_specs=[pl.BlockSpec((2500, 128), lambda: (0, 0))] * 2,
        out_specs=pl.BlockSpec((2500, 128), lambda: (0, 0)),
        out_shape=jax.ShapeDtypeStruct((2500, 128), jnp.int32),
    )(s2, e2).reshape(E)


def kernel(x, edge_index, edge_type, bases1, comb1, root1, bias1, g1, b1,
           bases2, comb2, root2, bias2, g2, b2):
    src = edge_index[0]
    dst = edge_index[1]
    gidx = _gidx_call(src, edge_type)

    xr1, hroot1 = _layer1_call(x, comb1, bases1, root1, bias1)
    xr1f = xr1.reshape(R * N, D)

    agg1 = _sc_max(xr1f, gidx, dst).reshape(NOUT, D)[:N]

    xr2, hroot2 = _mid_call(agg1, hroot1, g1, b1, comb2, bases2, root2, bias2)
    xr2f = xr2.reshape(R * N, D)

    zeros = jnp.zeros((NPS, D), jnp.float32)
    agg2 = _sc_sum(xr2f, gidx, dst, zeros)

    return _final_call(agg2[0], agg2[1], hroot2, g2, b2)
